# baseline (speedup 1.0000x reference)
import functools

import jax
import jax.numpy as jnp
from jax import lax
from jax.experimental import pallas as pl
from jax.experimental.pallas import tpu as pltpu
from jax.experimental.pallas import tpu_sc as plsc

NC = 2
NS = 16
NW = NC * NS

SLAB = 128
CHUNK_SLABS = 8
TILE_CHUNKS = 10


def _proj_body(x_ref, wn_ref, wi_ref, ws_ref, y_ref, z_ref, h_ref):
    xb = x_ref[...]
    y_ref[...] = jnp.dot(xb, wn_ref[...], preferred_element_type=jnp.float32)
    z_ref[...] = jnp.dot(xb, wi_ref[...], preferred_element_type=jnp.float32)
    h_ref[...] = jnp.dot(xb, ws_ref[...], preferred_element_type=jnp.float32)


def _combine_body(h_ref, nb_ref, cnt_ref, idp_ref, bias_ref, out_ref):
    ku = h_ref.shape[1]
    left = h_ref[...] + idp_ref[0] + idp_ref[1]
    cnt = cnt_ref[0, :, 0:1] + cnt_ref[1, :, 0:1]
    right = (nb_ref[0] + nb_ref[1]) / jnp.maximum(cnt, 1.0)
    bias = bias_ref[...]
    out_ref[:, 0:ku] = jax.nn.relu(left + bias[0, 0:ku])
    out_ref[:, ku:] = jax.nn.relu(right + bias[0, ku:])


def kernel(x, edge_index, id_index, W_self, W_id, W_neighbor, bias):
    n, d = x.shape
    ku = W_self.shape[1]
    e = edge_index.shape[1]
    nid = id_index.shape[0]

    chunk_e = SLAB * CHUNK_SLABS
    tile_slabs = TILE_CHUNKS * CHUNK_SLABS
    e_pad = NW * tile_slabs * SLAB
    n_slabs = e_pad // SLAB
    npad = n + 8
    id_pad = 8 * chunk_e
    id_chunks = id_pad // chunk_e
    rsub = 8 * ((n // NS) // 8)
    rlast = n - rsub * (NS - 1)

    blk = 2000
    grid1 = n // blk
    y, z, h = pl.pallas_call(
        _proj_body,
        grid=(grid1,),
        in_specs=[
            pl.BlockSpec((blk, d), lambda i: (i, 0)),
            pl.BlockSpec((d, ku), lambda i: (0, 0)),
            pl.BlockSpec((d, ku), lambda i: (0, 0)),
            pl.BlockSpec((d, ku), lambda i: (0, 0)),
        ],
        out_specs=[
            pl.BlockSpec((blk, ku), lambda i: (i, 0)),
            pl.BlockSpec((blk, ku), lambda i: (i, 0)),
            pl.BlockSpec((blk, ku), lambda i: (i, 0)),
        ],
        out_shape=[
            jax.ShapeDtypeStruct((n, ku), jnp.float32),
            jax.ShapeDtypeStruct((n, ku), jnp.float32),
            jax.ShapeDtypeStruct((n, ku), jnp.float32),
        ],
    )(x, W_neighbor, W_id, W_self)

    pad_e = jnp.full((e_pad - e,), n, dtype=jnp.int32)
    row2d = jnp.concatenate(
        [edge_index[0].astype(jnp.int32), pad_e]).reshape(n_slabs, SLAB)
    col2d = jnp.concatenate(
        [edge_index[1].astype(jnp.int32), pad_e]).reshape(n_slabs, SLAB)
    ids2d = jnp.concatenate(
        [id_index.astype(jnp.int32),
         jnp.full((id_pad - nid,), n, dtype=jnp.int32)]).reshape(
             id_pad // SLAB, SLAB)
    zrows = jnp.zeros((npad - n, ku), jnp.float32)
    ypad = jnp.concatenate([y, zrows])
    zpad = jnp.concatenate([z, zrows])
    zeros64 = jnp.zeros((rlast, ku), jnp.float32)
    zeros16 = jnp.zeros((rlast, 16), jnp.float32)
    ones16 = jnp.ones((SLAB, 16), jnp.float32)

    mesh = plsc.VectorSubcoreMesh(
        core_axis_name="c", subcore_axis_name="s",
        num_cores=NC, num_subcores=NS)

    @functools.partial(
        pl.kernel,
        out_type=(
            jax.ShapeDtypeStruct((NC, n, ku), jnp.float32),
            jax.ShapeDtypeStruct((NC, n, 16), jnp.float32),
            jax.ShapeDtypeStruct((NC, n, ku), jnp.float32),
        ),
        mesh=mesh,
        compiler_params=pltpu.CompilerParams(use_tc_tiling_on_sc=False),
        scratch_types=[
            pltpu.VMEM((CHUNK_SLABS, SLAB), jnp.int32),
            pltpu.VMEM((CHUNK_SLABS, SLAB), jnp.int32),
            pltpu.VMEM((CHUNK_SLABS, SLAB, ku), jnp.float32),
            pltpu.VMEM((SLAB, 16), jnp.float32),
            pltpu.VMEM_SHARED((npad, ku), jnp.float32),
            pltpu.VMEM_SHARED((npad, 16), jnp.float32),
            pltpu.SemaphoreType.DMA,
        ],
    )
    def sc_scatter(y_hbm, zp_hbm, row_hbm, col_hbm, ids_hbm, z64_hbm,
                   z16_hbm, ones_hbm, nb_out, cnt_out, idp_out,
                   row_v, col_v, rows_v, ones_v,
                   nb_acc, cnt_acc, sem):
        ci = lax.axis_index("c")
        si = lax.axis_index("s")
        wid = si * NC + ci
        slab0 = wid * tile_slabs

        r0 = si * rsub

        def zero_nb(nrows):
            pltpu.sync_copy(z64_hbm.at[pl.ds(0, nrows)],
                            nb_acc.at[pl.ds(r0, nrows)])

        @pl.when(si < NS - 1)
        def _():
            zero_nb(rsub)
            pltpu.sync_copy(z16_hbm.at[pl.ds(0, rsub)],
                            cnt_acc.at[pl.ds(r0, rsub)])

        @pl.when(si == NS - 1)
        def _():
            zero_nb(rlast)
            pltpu.sync_copy(z16_hbm.at[pl.ds(0, rlast)],
                            cnt_acc.at[pl.ds(r0, rlast)])

        pltpu.sync_copy(ones_hbm, ones_v)
        plsc.subcore_barrier()

        @pl.loop(0, TILE_CHUNKS)
        def _(k):
            base = slab0 + k * CHUNK_SLABS
            pltpu.sync_copy(row_hbm.at[pl.ds(base, CHUNK_SLABS)], row_v)
            pltpu.sync_copy(col_hbm.at[pl.ds(base, CHUNK_SLABS)], col_v)
            gdescs = []
            for j in range(CHUNK_SLABS):
                gdescs.append(pltpu.async_copy(
                    y_hbm.at[col_v.at[j]], rows_v.at[j], sem))
            for j in range(CHUNK_SLABS):
                gdescs[j].wait()
            for j in range(CHUNK_SLABS):
                pltpu.sync_copy(rows_v.at[j], nb_acc.at[row_v.at[j]],
                                add=True)
                pltpu.sync_copy(ones_v, cnt_acc.at[row_v.at[j]], add=True)

        plsc.subcore_barrier()

        def writeout(dst, nrows):
            sl = pl.ds(r0, nrows)
            pltpu.sync_copy(nb_acc.at[sl], dst.at[ci].at[sl])

        @pl.when(si < NS - 1)
        def _():
            writeout(nb_out, rsub)
            pltpu.sync_copy(cnt_acc.at[pl.ds(r0, rsub)],
                            cnt_out.at[ci].at[pl.ds(r0, rsub)])
            zero_nb(rsub)

        @pl.when(si == NS - 1)
        def _():
            writeout(nb_out, rlast)
            pltpu.sync_copy(cnt_acc.at[pl.ds(r0, rlast)],
                            cnt_out.at[ci].at[pl.ds(r0, rlast)])
            zero_nb(rlast)

        plsc.subcore_barrier()

        @pl.when(wid < id_chunks)
        def _():
            base = wid * CHUNK_SLABS
            pltpu.sync_copy(ids_hbm.at[pl.ds(base, CHUNK_SLABS)], row_v)
            gdescs = []
            for j in range(CHUNK_SLABS):
                gdescs.append(pltpu.async_copy(
                    zp_hbm.at[row_v.at[j]], rows_v.at[j], sem))
            for j in range(CHUNK_SLABS):
                gdescs[j].wait()
            for j in range(CHUNK_SLABS):
                pltpu.sync_copy(rows_v.at[j], nb_acc.at[row_v.at[j]],
                                add=True)

        plsc.subcore_barrier()

        @pl.when(si < NS - 1)
        def _():
            writeout(idp_out, rsub)

        @pl.when(si == NS - 1)
        def _():
            writeout(idp_out, rlast)

    nb_p, cnt_p, idp_p = sc_scatter(ypad, zpad, row2d, col2d, ids2d,
                                    zeros64, zeros16, ones16)

    out = pl.pallas_call(
        _combine_body,
        grid=(grid1,),
        in_specs=[
            pl.BlockSpec((blk, ku), lambda i: (i, 0)),
            pl.BlockSpec((NC, blk, ku), lambda i: (0, i, 0)),
            pl.BlockSpec((NC, blk, 16), lambda i: (0, i, 0)),
            pl.BlockSpec((NC, blk, ku), lambda i: (0, i, 0)),
            pl.BlockSpec((1, 2 * ku), lambda i: (0, 0)),
        ],
        out_specs=pl.BlockSpec((blk, 2 * ku), lambda i: (i, 0)),
        out_shape=jax.ShapeDtypeStruct((n, 2 * ku), jnp.float32),
    )(h, nb_p, cnt_p, idp_p, bias.reshape(1, 2 * ku))
    return out


# speedup vs baseline: 2.4386x; 2.4386x over previous
import functools

import jax
import jax.numpy as jnp
from jax import lax
from jax.experimental import pallas as pl
from jax.experimental.pallas import tpu as pltpu
from jax.experimental.pallas import tpu_sc as plsc

NC = 2
NS = 16
NW = NC * NS

SLAB = 128
CHUNK_SLABS = 8
CHUNK_E = SLAB * CHUNK_SLABS


def _proj_body(x_ref, wn_ref, wi_ref, ws_ref, y_ref, z_ref, h_ref):
    xb = x_ref[...]
    y_ref[...] = jnp.dot(xb, wn_ref[...], preferred_element_type=jnp.float32)
    z_ref[...] = jnp.dot(xb, wi_ref[...], preferred_element_type=jnp.float32)
    h_ref[...] = jnp.dot(xb, ws_ref[...], preferred_element_type=jnp.float32)


def _combine_body(h_ref, nb_ref, cnt_ref, idp_ref, bias_ref, out_ref):
    ku = h_ref.shape[1]
    blk = h_ref.shape[0]
    left = h_ref[...] + idp_ref[0] + idp_ref[1]
    cnt = jnp.sum(cnt_ref[...], axis=1)
    right = (nb_ref[0] + nb_ref[1]) / jnp.maximum(cnt[:, None], 1.0)
    bias = bias_ref[...]
    out_ref[:, 0:ku] = jax.nn.relu(left + bias[0, 0:ku])
    out_ref[:, ku:] = jax.nn.relu(right + bias[0, ku:])


def kernel(x, edge_index, id_index, W_self, W_id, W_neighbor, bias):
    n, d = x.shape
    ku = W_self.shape[1]
    e = edge_index.shape[1]
    nid = id_index.shape[0]

    n_chunks = e // CHUNK_E
    tail_e = e - n_chunks * CHUNK_E
    chunks_per_tile = -(-n_chunks // NW)
    npad = n + 8
    id_pad = -(-nid // SLAB) * SLAB
    id_chunks = -(-(id_pad // SLAB) // CHUNK_SLABS)
    rsub = 8 * ((n // NS) // 8)
    rlast = n - rsub * (NS - 1)

    blk = 2000
    grid1 = n // blk
    y, z, h = pl.pallas_call(
        _proj_body,
        grid=(grid1,),
        in_specs=[
            pl.BlockSpec((blk, d), lambda i: (i, 0)),
            pl.BlockSpec((d, ku), lambda i: (0, 0)),
            pl.BlockSpec((d, ku), lambda i: (0, 0)),
            pl.BlockSpec((d, ku), lambda i: (0, 0)),
        ],
        out_specs=[
            pl.BlockSpec((blk, ku), lambda i: (i, 0)),
            pl.BlockSpec((blk, ku), lambda i: (i, 0)),
            pl.BlockSpec((blk, ku), lambda i: (i, 0)),
        ],
        out_shape=[
            jax.ShapeDtypeStruct((n, ku), jnp.float32),
            jax.ShapeDtypeStruct((n, ku), jnp.float32),
            jax.ShapeDtypeStruct((n, ku), jnp.float32),
        ],
    )(x, W_neighbor, W_id, W_self)

    row1d = edge_index[0].astype(jnp.int32)
    col1d = edge_index[1].astype(jnp.int32)
    ids1d = jnp.concatenate(
        [id_index.astype(jnp.int32),
         jnp.full((id_pad - nid,), n, dtype=jnp.int32)])
    zpad = jnp.concatenate([z, jnp.zeros((npad - n, ku), jnp.float32)])
    zeros64 = jnp.zeros((rlast, ku), jnp.float32)

    mesh = plsc.VectorSubcoreMesh(
        core_axis_name="c", subcore_axis_name="s",
        num_cores=NC, num_subcores=NS)

    @functools.partial(
        pl.kernel,
        out_type=(
            jax.ShapeDtypeStruct((NC, n, ku), jnp.float32),
            jax.ShapeDtypeStruct((NW, n), jnp.float32),
            jax.ShapeDtypeStruct((NC, n, ku), jnp.float32),
        ),
        mesh=mesh,
        compiler_params=pltpu.CompilerParams(
            use_tc_tiling_on_sc=False, needs_layout_passes=False),
        scratch_types=[
            pltpu.VMEM((CHUNK_E,), jnp.int32),
            pltpu.VMEM((CHUNK_E,), jnp.int32),
            pltpu.VMEM((CHUNK_E, ku), jnp.float32),
            pltpu.VMEM((n,), jnp.float32),
            pltpu.VMEM_SHARED((npad, ku), jnp.float32),
            pltpu.SemaphoreType.DMA,
        ],
    )
    def sc_scatter(y_hbm, zp_hbm, row_hbm, col_hbm, ids_hbm, z64_hbm,
                   nb_out, cnt_out, idp_out,
                   row_v, col_v, rows_v, hist_v,
                   nb_acc, sem):
        ci = lax.axis_index("c")
        si = lax.axis_index("s")
        wid = si * NC + ci

        r0 = si * rsub

        def zero_nb(nrows):
            pltpu.sync_copy(z64_hbm.at[pl.ds(0, nrows)],
                            nb_acc.at[pl.ds(r0, nrows)])

        @pl.when(si < NS - 1)
        def _():
            zero_nb(rsub)

        @pl.when(si == NS - 1)
        def _():
            zero_nb(rlast)

        zv = jnp.zeros((16,), jnp.float32)

        @pl.loop(0, n, step=16)
        def _(i):
            hist_v[pl.ds(i, 16)] = zv

        plsc.subcore_barrier()

        ones_reg = jnp.ones((16,), jnp.float32)

        def do_edges(ne):
            if ne == CHUNK_E:
                rv, cv, gv = row_v, col_v, rows_v
            else:
                sl = pl.ds(0, ne)
                rv, cv = row_v.at[sl], col_v.at[sl]
                gv = rows_v.at[sl]
            gd = pltpu.async_copy(y_hbm.at[cv], gv, sem)

            @pl.loop(0, ne, step=16)
            def _(i):
                plsc.addupdate_scatter(hist_v, [row_v[pl.ds(i, 16)]],
                                       ones_reg)

            gd.wait()
            pltpu.sync_copy(gv, nb_acc.at[rv], add=True)

        @pl.loop(0, chunks_per_tile)
        def _(k):
            c = wid + k * NW

            @pl.when(c < n_chunks)
            def _():
                base = c * CHUNK_E
                pltpu.sync_copy(row_hbm.at[pl.ds(base, CHUNK_E)], row_v)
                pltpu.sync_copy(col_hbm.at[pl.ds(base, CHUNK_E)], col_v)
                do_edges(CHUNK_E)

        @pl.when(wid == NW - 1)
        def _():
            base = n_chunks * CHUNK_E
            pltpu.sync_copy(row_hbm.at[pl.ds(base, tail_e)],
                            row_v.at[pl.ds(0, tail_e)])
            pltpu.sync_copy(col_hbm.at[pl.ds(base, tail_e)],
                            col_v.at[pl.ds(0, tail_e)])
            do_edges(tail_e)

        plsc.subcore_barrier()

        def writeout(dst, nrows):
            sl = pl.ds(r0, nrows)
            pltpu.sync_copy(nb_acc.at[sl], dst.at[ci].at[sl])

        @pl.when(si < NS - 1)
        def _():
            writeout(nb_out, rsub)
            pltpu.sync_copy(hist_v, cnt_out.at[wid])
            zero_nb(rsub)

        @pl.when(si == NS - 1)
        def _():
            writeout(nb_out, rlast)
            pltpu.sync_copy(hist_v, cnt_out.at[wid])
            zero_nb(rlast)

        plsc.subcore_barrier()

        @pl.when(wid < id_chunks)
        def _():
            base = wid * CHUNK_E
            pltpu.sync_copy(ids_hbm.at[pl.ds(base, CHUNK_E)], row_v)
            pltpu.async_copy(zp_hbm.at[row_v], rows_v, sem).wait()
            pltpu.sync_copy(rows_v, nb_acc.at[row_v], add=True)

        plsc.subcore_barrier()

        @pl.when(si < NS - 1)
        def _():
            writeout(idp_out, rsub)

        @pl.when(si == NS - 1)
        def _():
            writeout(idp_out, rlast)

    nb_p, cnt_p, idp_p = sc_scatter(y, zpad, row1d, col1d, ids1d, zeros64)

    out = pl.pallas_call(
        _combine_body,
        grid=(grid1,),
        in_specs=[
            pl.BlockSpec((blk, ku), lambda i: (i, 0)),
            pl.BlockSpec((NC, blk, ku), lambda i: (0, i, 0)),
            pl.BlockSpec((blk, NW), lambda i: (i, 0)),
            pl.BlockSpec((NC, blk, ku), lambda i: (0, i, 0)),
            pl.BlockSpec((1, 2 * ku), lambda i: (0, 0)),
        ],
        out_specs=pl.BlockSpec((blk, 2 * ku), lambda i: (i, 0)),
        out_shape=jax.ShapeDtypeStruct((n, 2 * ku), jnp.float32),
    )(h, nb_p, cnt_p.T, idp_p, bias.reshape(1, 2 * ku))
    return out


# speedup vs baseline: 2.8691x; 1.1765x over previous
import functools

import jax
import jax.numpy as jnp
from jax import lax
from jax.experimental import pallas as pl
from jax.experimental.pallas import tpu as pltpu
from jax.experimental.pallas import tpu_sc as plsc

NC = 2
NS = 16
NW = NC * NS

SLAB = 128
CHUNK_E = 512
ID_CHUNK = 512


def _proj_body(x_ref, wn_ref, wi_ref, ws_ref, y_ref, z_ref, h_ref):
    xb = x_ref[...]
    y_ref[...] = jnp.dot(xb, wn_ref[...], preferred_element_type=jnp.float32)
    z_ref[...] = jnp.dot(xb, wi_ref[...], preferred_element_type=jnp.float32)
    h_ref[...] = jnp.dot(xb, ws_ref[...], preferred_element_type=jnp.float32)


def _combine_body(h_ref, nb_ref, cnt_ref, idp_ref, bias_ref, out_ref):
    ku = h_ref.shape[1]
    blk = h_ref.shape[0]
    left = h_ref[...] + idp_ref[0] + idp_ref[1]
    cnt = jnp.sum(cnt_ref[...], axis=1)
    right = (nb_ref[0] + nb_ref[1]) / jnp.maximum(cnt[:, None], 1.0)
    bias = bias_ref[...]
    out_ref[:, 0:ku] = jax.nn.relu(left + bias[0, 0:ku])
    out_ref[:, ku:] = jax.nn.relu(right + bias[0, ku:])


def kernel(x, edge_index, id_index, W_self, W_id, W_neighbor, bias):
    n, d = x.shape
    ku = W_self.shape[1]
    e = edge_index.shape[1]
    nid = id_index.shape[0]

    tile_e = e // NW
    full_chunks = tile_e // CHUNK_E
    tail_e = tile_e - full_chunks * CHUNK_E
    n_bodies = full_chunks + (1 if tail_e else 0)
    npad = n + 8
    id_pad = -(-nid // ID_CHUNK) * ID_CHUNK
    id_chunks = id_pad // ID_CHUNK
    rsub = 8 * ((n // NS) // 8)
    rlast = n - rsub * (NS - 1)

    blk = 2000
    grid1 = n // blk
    y, z, h = pl.pallas_call(
        _proj_body,
        grid=(grid1,),
        in_specs=[
            pl.BlockSpec((blk, d), lambda i: (i, 0)),
            pl.BlockSpec((d, ku), lambda i: (0, 0)),
            pl.BlockSpec((d, ku), lambda i: (0, 0)),
            pl.BlockSpec((d, ku), lambda i: (0, 0)),
        ],
        out_specs=[
            pl.BlockSpec((blk, ku), lambda i: (i, 0)),
            pl.BlockSpec((blk, ku), lambda i: (i, 0)),
            pl.BlockSpec((blk, ku), lambda i: (i, 0)),
        ],
        out_shape=[
            jax.ShapeDtypeStruct((n, ku), jnp.float32),
            jax.ShapeDtypeStruct((n, ku), jnp.float32),
            jax.ShapeDtypeStruct((n, ku), jnp.float32),
        ],
    )(x, W_neighbor, W_id, W_self)

    row1d = edge_index[0].astype(jnp.int32)
    col1d = edge_index[1].astype(jnp.int32)
    ids1d = jnp.concatenate(
        [id_index.astype(jnp.int32),
         jnp.full((id_pad - nid,), n, dtype=jnp.int32)])
    zpad = jnp.concatenate([z, jnp.zeros((npad - n, ku), jnp.float32)])
    zeros64 = jnp.zeros((rlast, ku), jnp.float32)

    mesh = plsc.VectorSubcoreMesh(
        core_axis_name="c", subcore_axis_name="s",
        num_cores=NC, num_subcores=NS)

    @functools.partial(
        pl.kernel,
        out_type=(
            jax.ShapeDtypeStruct((NC, n, ku), jnp.float32),
            jax.ShapeDtypeStruct((NW, n), jnp.float32),
            jax.ShapeDtypeStruct((NC, n, ku), jnp.float32),
        ),
        mesh=mesh,
        compiler_params=pltpu.CompilerParams(
            use_tc_tiling_on_sc=False, needs_layout_passes=False),
        scratch_types=[
            pltpu.VMEM((CHUNK_E,), jnp.int32),
            pltpu.VMEM((CHUNK_E,), jnp.int32),
            pltpu.VMEM((CHUNK_E,), jnp.int32),
            pltpu.VMEM((CHUNK_E,), jnp.int32),
            pltpu.VMEM((CHUNK_E, ku), jnp.float32),
            pltpu.VMEM((CHUNK_E, ku), jnp.float32),
            pltpu.VMEM((n,), jnp.float32),
            pltpu.VMEM_SHARED((npad, ku), jnp.float32),
            pltpu.SemaphoreType.DMA,
            pltpu.SemaphoreType.DMA,
            pltpu.SemaphoreType.DMA,
            pltpu.SemaphoreType.DMA,
        ],
    )
    def sc_scatter(y_hbm, zp_hbm, row_hbm, col_hbm, ids_hbm, z64_hbm,
                   nb_out, cnt_out, idp_out,
                   row_v0, row_v1, col_v0, col_v1, rows_v0, rows_v1, hist_v,
                   nb_acc, sem_i, sem_g0, sem_g1, sem_s):
        ci = lax.axis_index("c")
        si = lax.axis_index("s")
        wid = si * NC + ci

        r0 = si * rsub

        def zero_nb(nrows):
            pltpu.sync_copy(z64_hbm.at[pl.ds(0, nrows)],
                            nb_acc.at[pl.ds(r0, nrows)])

        @pl.when(si < NS - 1)
        def _():
            zero_nb(rsub)

        @pl.when(si == NS - 1)
        def _():
            zero_nb(rlast)

        zv = jnp.zeros((16,), jnp.float32)

        @pl.loop(0, n, step=16)
        def _(i):
            hist_v[pl.ds(i, 16)] = zv

        plsc.subcore_barrier()

        ones_reg = jnp.ones((16,), jnp.float32)
        e0 = wid * tile_e
        row_b = (row_v0, row_v1)
        col_b = (col_v0, col_v1)
        rows_b = (rows_v0, rows_v1)
        gsem = (sem_g0, sem_g1)
        sizes = [CHUNK_E] * full_chunks + ([tail_e] if tail_e else [])

        def fire_idx(k, p):
            base = e0 + k * CHUNK_E
            ne = sizes[k]
            pltpu.async_copy(row_hbm.at[pl.ds(base, ne)],
                             row_b[p].at[pl.ds(0, ne)], sem_i)
            pltpu.async_copy(col_hbm.at[pl.ds(base, ne)],
                             col_b[p].at[pl.ds(0, ne)], sem_i)

        fire_idx(0, 0)
        prev_s = None
        for k in range(n_bodies):
            p = k % 2
            ne = sizes[k]
            sl = pl.ds(0, ne)
            rv = row_b[p].at[sl]
            cv = col_b[p].at[sl]
            gv = rows_b[p].at[sl]
            pltpu.make_async_copy(
                row_hbm.at[pl.ds(e0, ne)], rv, sem_i).wait()
            pltpu.make_async_copy(
                col_hbm.at[pl.ds(e0, ne)], cv, sem_i).wait()
            gd = pltpu.async_copy(y_hbm.at[cv], gv, gsem[p])
            if prev_s is not None:
                prev_s.wait()
            if k + 1 < n_bodies:
                fire_idx(k + 1, 1 - p)
            @pl.loop(0, ne, step=16)
            def _(i, _p=p):
                plsc.addupdate_scatter(
                    hist_v, [row_b[_p][pl.ds(i, 16)]], ones_reg)

            gd.wait()
            prev_s = pltpu.async_copy(gv, nb_acc.at[rv], sem_s, add=True)
        prev_s.wait()

        plsc.subcore_barrier()

        def writeout(dst, nrows):
            sl = pl.ds(r0, nrows)
            pltpu.sync_copy(nb_acc.at[sl], dst.at[ci].at[sl])

        @pl.when(si < NS - 1)
        def _():
            writeout(nb_out, rsub)
            pltpu.sync_copy(hist_v, cnt_out.at[wid])
            zero_nb(rsub)

        @pl.when(si == NS - 1)
        def _():
            writeout(nb_out, rlast)
            pltpu.sync_copy(hist_v, cnt_out.at[wid])
            zero_nb(rlast)

        plsc.subcore_barrier()

        @pl.when(wid < id_chunks)
        def _():
            base = wid * ID_CHUNK
            pltpu.sync_copy(ids_hbm.at[pl.ds(base, ID_CHUNK)], row_v0)
            pltpu.async_copy(zp_hbm.at[row_v0], rows_v0, sem_g0).wait()
            pltpu.sync_copy(rows_v0, nb_acc.at[row_v0], add=True)

        plsc.subcore_barrier()

        @pl.when(si < NS - 1)
        def _():
            writeout(idp_out, rsub)

        @pl.when(si == NS - 1)
        def _():
            writeout(idp_out, rlast)

    nb_p, cnt_p, idp_p = sc_scatter(y, zpad, row1d, col1d, ids1d, zeros64)

    out = pl.pallas_call(
        _combine_body,
        grid=(grid1,),
        in_specs=[
            pl.BlockSpec((blk, ku), lambda i: (i, 0)),
            pl.BlockSpec((NC, blk, ku), lambda i: (0, i, 0)),
            pl.BlockSpec((blk, NW), lambda i: (i, 0)),
            pl.BlockSpec((NC, blk, ku), lambda i: (0, i, 0)),
            pl.BlockSpec((1, 2 * ku), lambda i: (0, 0)),
        ],
        out_specs=pl.BlockSpec((blk, 2 * ku), lambda i: (i, 0)),
        out_shape=jax.ShapeDtypeStruct((n, 2 * ku), jnp.float32),
    )(h, nb_p, cnt_p.T, idp_p, bias.reshape(1, 2 * ku))
    return out


# speedup vs baseline: 2.9362x; 1.0234x over previous
import functools

import jax
import jax.numpy as jnp
from jax import lax
from jax.experimental import pallas as pl
from jax.experimental.pallas import tpu as pltpu
from jax.experimental.pallas import tpu_sc as plsc

NC = 2
NS = 16
NW = NC * NS

SLAB = 128
CHUNK_E = 512
ID_CHUNK = 512


def _proj_body(x_ref, wn_ref, wi_ref, ws_ref, y_ref, z_ref, h_ref):
    xb = x_ref[...]
    y_ref[...] = jnp.dot(xb, wn_ref[...], preferred_element_type=jnp.float32)
    z_ref[...] = jnp.dot(xb, wi_ref[...], preferred_element_type=jnp.float32)
    h_ref[...] = jnp.dot(xb, ws_ref[...], preferred_element_type=jnp.float32)


def _combine_body(h_ref, nb_ref, cnt_ref, idp_ref, bias_ref, out_ref):
    ku = h_ref.shape[1]
    blk = h_ref.shape[0]
    left = h_ref[...] + idp_ref[0] + idp_ref[1]
    cnt = jnp.sum(cnt_ref[...], axis=1)
    right = (nb_ref[0] + nb_ref[1]) / jnp.maximum(cnt[:, None], 1.0)
    bias = bias_ref[...]
    out_ref[:, 0:ku] = jax.nn.relu(left + bias[0, 0:ku])
    out_ref[:, ku:] = jax.nn.relu(right + bias[0, ku:])


def kernel(x, edge_index, id_index, W_self, W_id, W_neighbor, bias):
    n, d = x.shape
    ku = W_self.shape[1]
    e = edge_index.shape[1]
    nid = id_index.shape[0]

    tile_e = e // NW
    full_chunks = tile_e // CHUNK_E
    tail_e = tile_e - full_chunks * CHUNK_E
    n_bodies = full_chunks + (1 if tail_e else 0)
    npad = n + 8
    id_pad = -(-nid // ID_CHUNK) * ID_CHUNK
    id_chunks = id_pad // ID_CHUNK
    rsub = 8 * ((n // NS) // 8)
    rlast = n - rsub * (NS - 1)

    blk = 2000
    grid1 = n // blk
    y, z, h = pl.pallas_call(
        _proj_body,
        grid=(grid1,),
        in_specs=[
            pl.BlockSpec((blk, d), lambda i: (i, 0)),
            pl.BlockSpec((d, ku), lambda i: (0, 0)),
            pl.BlockSpec((d, ku), lambda i: (0, 0)),
            pl.BlockSpec((d, ku), lambda i: (0, 0)),
        ],
        out_specs=[
            pl.BlockSpec((blk, ku), lambda i: (i, 0)),
            pl.BlockSpec((blk, ku), lambda i: (i, 0)),
            pl.BlockSpec((blk, ku), lambda i: (i, 0)),
        ],
        out_shape=[
            jax.ShapeDtypeStruct((n, ku), jnp.float32),
            jax.ShapeDtypeStruct((n, ku), jnp.float32),
            jax.ShapeDtypeStruct((n, ku), jnp.float32),
        ],
    )(x, W_neighbor, W_id, W_self)

    row1d = edge_index[0].astype(jnp.int32)
    col1d = edge_index[1].astype(jnp.int32)
    ids1d = jnp.concatenate(
        [id_index.astype(jnp.int32),
         jnp.full((id_pad - nid,), n, dtype=jnp.int32)])
    zpad = jnp.concatenate([z, jnp.zeros((npad - n, ku), jnp.float32)])
    zeros64 = jnp.zeros((rlast, ku), jnp.float32)

    mesh = plsc.VectorSubcoreMesh(
        core_axis_name="c", subcore_axis_name="s",
        num_cores=NC, num_subcores=NS)

    @functools.partial(
        pl.kernel,
        out_type=(
            jax.ShapeDtypeStruct((NC, n, ku), jnp.float32),
            jax.ShapeDtypeStruct((NW, n), jnp.float32),
            jax.ShapeDtypeStruct((NC, n, ku), jnp.float32),
        ),
        mesh=mesh,
        compiler_params=pltpu.CompilerParams(
            use_tc_tiling_on_sc=False, needs_layout_passes=False),
        scratch_types=[
            pltpu.VMEM((4, CHUNK_E), jnp.int32),
            pltpu.VMEM((4, CHUNK_E), jnp.int32),
            pltpu.VMEM((CHUNK_E, ku), jnp.float32),
            pltpu.VMEM((CHUNK_E, ku), jnp.float32),
            pltpu.VMEM((n,), jnp.float32),
            pltpu.VMEM_SHARED((npad, ku), jnp.float32),
            pltpu.SemaphoreType.DMA,
            pltpu.SemaphoreType.DMA,
            pltpu.SemaphoreType.DMA,
            pltpu.SemaphoreType.DMA,
        ],
    )
    def sc_scatter(y_hbm, zp_hbm, row_hbm, col_hbm, ids_hbm, z64_hbm,
                   nb_out, cnt_out, idp_out,
                   row_r, col_r, rows_v0, rows_v1, hist_v,
                   nb_acc, sem_i, sem_g0, sem_g1, sem_s):
        ci = lax.axis_index("c")
        si = lax.axis_index("s")
        wid = si * NC + ci

        r0 = si * rsub

        def zero_nb(nrows):
            pltpu.sync_copy(z64_hbm.at[pl.ds(0, nrows)],
                            nb_acc.at[pl.ds(r0, nrows)])

        @pl.when(si < NS - 1)
        def _():
            zero_nb(rsub)

        @pl.when(si == NS - 1)
        def _():
            zero_nb(rlast)

        zv = jnp.zeros((16,), jnp.float32)

        @pl.loop(0, n, step=16)
        def _(i):
            hist_v[pl.ds(i, 16)] = zv

        plsc.subcore_barrier()

        ones_reg = jnp.ones((16,), jnp.float32)
        e0 = wid * tile_e
        rows_b = (rows_v0, rows_v1)
        gsem = (sem_g0, sem_g1)
        sizes = [CHUNK_E] * full_chunks + ([tail_e] if tail_e else [])

        def idx_refs(k):
            ne = sizes[k]
            sl = pl.ds(0, ne)
            return row_r.at[k % 4].at[sl], col_r.at[k % 4].at[sl]

        def fire_idx(k):
            base = e0 + k * CHUNK_E
            ne = sizes[k]
            rv, cv = idx_refs(k)
            pltpu.async_copy(row_hbm.at[pl.ds(base, ne)], rv, sem_i)
            pltpu.async_copy(col_hbm.at[pl.ds(base, ne)], cv, sem_i)

        def wait_idx(k):
            ne = sizes[k]
            rv, cv = idx_refs(k)
            pltpu.make_async_copy(row_hbm.at[pl.ds(e0, ne)], rv,
                                  sem_i).wait()
            pltpu.make_async_copy(col_hbm.at[pl.ds(e0, ne)], cv,
                                  sem_i).wait()

        def fire_gather(k):
            ne = sizes[k]
            _, cv = idx_refs(k)
            return pltpu.async_copy(
                y_hbm.at[cv], rows_b[k % 2].at[pl.ds(0, ne)], gsem[k % 2])

        fire_idx(0)
        if n_bodies > 1:
            fire_idx(1)
        wait_idx(0)
        gds = {0: fire_gather(0)}
        sds = {}
        for k in range(n_bodies):
            p = k % 2
            ne = sizes[k]
            rv, _ = idx_refs(k)
            if k - 1 in sds:
                sds[k - 1].wait()
            if k + 1 < n_bodies:
                wait_idx(k + 1)
                gds[k + 1] = fire_gather(k + 1)
            if k + 2 < n_bodies:
                fire_idx(k + 2)
            @pl.loop(0, ne, step=16)
            def _(i, _k=k):
                plsc.addupdate_scatter(
                    hist_v, [row_r.at[_k % 4][pl.ds(i, 16)]], ones_reg)

            gds[k].wait()
            sds[k] = pltpu.async_copy(rows_b[p].at[pl.ds(0, ne)],
                                      nb_acc.at[rv], sem_s, add=True)
        sds[n_bodies - 1].wait()

        plsc.subcore_barrier()

        def writeout(dst, nrows):
            sl = pl.ds(r0, nrows)
            pltpu.sync_copy(nb_acc.at[sl], dst.at[ci].at[sl])

        @pl.when(si < NS - 1)
        def _():
            writeout(nb_out, rsub)
            pltpu.sync_copy(hist_v, cnt_out.at[wid])
            zero_nb(rsub)

        @pl.when(si == NS - 1)
        def _():
            writeout(nb_out, rlast)
            pltpu.sync_copy(hist_v, cnt_out.at[wid])
            zero_nb(rlast)

        plsc.subcore_barrier()

        @pl.when(wid < id_chunks)
        def _():
            base = wid * ID_CHUNK
            idv = row_r.at[0]
            pltpu.sync_copy(ids_hbm.at[pl.ds(base, ID_CHUNK)], idv)
            pltpu.async_copy(zp_hbm.at[idv], rows_v0, sem_g0).wait()
            pltpu.sync_copy(rows_v0, nb_acc.at[idv], add=True)

        plsc.subcore_barrier()

        @pl.when(si < NS - 1)
        def _():
            writeout(idp_out, rsub)

        @pl.when(si == NS - 1)
        def _():
            writeout(idp_out, rlast)

    nb_p, cnt_p, idp_p = sc_scatter(y, zpad, row1d, col1d, ids1d, zeros64)

    out = pl.pallas_call(
        _combine_body,
        grid=(grid1,),
        in_specs=[
            pl.BlockSpec((blk, ku), lambda i: (i, 0)),
            pl.BlockSpec((NC, blk, ku), lambda i: (0, i, 0)),
            pl.BlockSpec((blk, NW), lambda i: (i, 0)),
            pl.BlockSpec((NC, blk, ku), lambda i: (0, i, 0)),
            pl.BlockSpec((1, 2 * ku), lambda i: (0, 0)),
        ],
        out_specs=pl.BlockSpec((blk, 2 * ku), lambda i: (i, 0)),
        out_shape=jax.ShapeDtypeStruct((n, 2 * ku), jnp.float32),
    )(h, nb_p, cnt_p.T, idp_p, bias.reshape(1, 2 * ku))
    return out


# speedup vs baseline: 3.0023x; 1.0225x over previous
import functools

import jax
import jax.numpy as jnp
from jax import lax
from jax.experimental import pallas as pl
from jax.experimental.pallas import tpu as pltpu
from jax.experimental.pallas import tpu_sc as plsc

NC = 2
NS = 16
NW = NC * NS

SLAB = 128
CHUNK_E = 576
ID_CHUNK = 512


def _proj_body(x_ref, wn_ref, wi_ref, ws_ref, y_ref, z_ref, h_ref):
    xb = x_ref[...]
    y_ref[...] = jnp.dot(xb, wn_ref[...], preferred_element_type=jnp.float32)
    z_ref[...] = jnp.dot(xb, wi_ref[...], preferred_element_type=jnp.float32)
    h_ref[...] = jnp.dot(xb, ws_ref[...], preferred_element_type=jnp.float32)


def _combine_body(h_ref, nb_ref, cnt_ref, idp_ref, bias_ref, out_ref):
    ku = h_ref.shape[1]
    blk = h_ref.shape[0]
    left = h_ref[...] + idp_ref[0] + idp_ref[1]
    cnt = jnp.sum(cnt_ref[...], axis=1)
    right = (nb_ref[0] + nb_ref[1]) / jnp.maximum(cnt[:, None], 1.0)
    bias = bias_ref[...]
    out_ref[:, 0:ku] = jax.nn.relu(left + bias[0, 0:ku])
    out_ref[:, ku:] = jax.nn.relu(right + bias[0, ku:])


def kernel(x, edge_index, id_index, W_self, W_id, W_neighbor, bias):
    n, d = x.shape
    ku = W_self.shape[1]
    e = edge_index.shape[1]
    nid = id_index.shape[0]

    tile_e = e // NW
    full_chunks = tile_e // CHUNK_E
    tail_e = tile_e - full_chunks * CHUNK_E
    n_bodies = full_chunks + (1 if tail_e else 0)
    npad = n + 8
    id_pad = -(-nid // ID_CHUNK) * ID_CHUNK
    id_chunks = id_pad // ID_CHUNK
    rsub = 8 * ((n // NS) // 8)
    rlast = n - rsub * (NS - 1)

    blk = 2000
    grid1 = n // blk
    y, z, h = pl.pallas_call(
        _proj_body,
        grid=(grid1,),
        in_specs=[
            pl.BlockSpec((blk, d), lambda i: (i, 0)),
            pl.BlockSpec((d, ku), lambda i: (0, 0)),
            pl.BlockSpec((d, ku), lambda i: (0, 0)),
            pl.BlockSpec((d, ku), lambda i: (0, 0)),
        ],
        out_specs=[
            pl.BlockSpec((blk, ku), lambda i: (i, 0)),
            pl.BlockSpec((blk, ku), lambda i: (i, 0)),
            pl.BlockSpec((blk, ku), lambda i: (i, 0)),
        ],
        out_shape=[
            jax.ShapeDtypeStruct((n, ku), jnp.float32),
            jax.ShapeDtypeStruct((n, ku), jnp.float32),
            jax.ShapeDtypeStruct((n, ku), jnp.float32),
        ],
    )(x, W_neighbor, W_id, W_self)

    row1d = edge_index[0].astype(jnp.int32)
    col1d = edge_index[1].astype(jnp.int32)
    ids1d = jnp.concatenate(
        [id_index.astype(jnp.int32),
         jnp.full((id_pad - nid,), n, dtype=jnp.int32)])
    zpad = jnp.concatenate([z, jnp.zeros((npad - n, ku), jnp.float32)])
    zeros64 = jnp.zeros((rlast, ku), jnp.float32)

    mesh = plsc.VectorSubcoreMesh(
        core_axis_name="c", subcore_axis_name="s",
        num_cores=NC, num_subcores=NS)

    @functools.partial(
        pl.kernel,
        out_type=(
            jax.ShapeDtypeStruct((NC, n, ku), jnp.float32),
            jax.ShapeDtypeStruct((NW, n), jnp.float32),
            jax.ShapeDtypeStruct((NC, n, ku), jnp.float32),
        ),
        mesh=mesh,
        compiler_params=pltpu.CompilerParams(
            use_tc_tiling_on_sc=False, needs_layout_passes=False),
        scratch_types=[
            pltpu.VMEM((4, CHUNK_E), jnp.int32),
            pltpu.VMEM((4, CHUNK_E), jnp.int32),
            pltpu.VMEM((CHUNK_E, ku), jnp.float32),
            pltpu.VMEM((CHUNK_E, ku), jnp.float32),
            pltpu.VMEM((n,), jnp.float32),
            pltpu.VMEM_SHARED((npad, ku), jnp.float32),
            pltpu.SemaphoreType.DMA,
            pltpu.SemaphoreType.DMA,
            pltpu.SemaphoreType.DMA,
            pltpu.SemaphoreType.DMA,
        ],
    )
    def sc_scatter(y_hbm, zp_hbm, row_hbm, col_hbm, ids_hbm, z64_hbm,
                   nb_out, cnt_out, idp_out,
                   row_r, col_r, rows_v0, rows_v1, hist_v,
                   nb_acc, sem_i, sem_g0, sem_g1, sem_s):
        ci = lax.axis_index("c")
        si = lax.axis_index("s")
        wid = si * NC + ci

        r0 = si * rsub

        def zero_nb_async(nrows):
            return pltpu.async_copy(z64_hbm.at[pl.ds(0, nrows)],
                                    nb_acc.at[pl.ds(r0, nrows)], sem_s)

        def zero_nb(nrows):
            zero_nb_async(nrows).wait()

        @pl.when(si < NS - 1)
        def _():
            zd = zero_nb_async(rsub)
            zv = jnp.zeros((16,), jnp.float32)

            @pl.loop(0, n, step=16)
            def _(i):
                hist_v[pl.ds(i, 16)] = zv

            zd.wait()

        @pl.when(si == NS - 1)
        def _():
            zd = zero_nb_async(rlast)
            zv = jnp.zeros((16,), jnp.float32)

            @pl.loop(0, n, step=16)
            def _(i):
                hist_v[pl.ds(i, 16)] = zv

            zd.wait()

        plsc.subcore_barrier()

        ones_reg = jnp.ones((16,), jnp.float32)
        e0 = wid * tile_e
        rows_b = (rows_v0, rows_v1)
        gsem = (sem_g0, sem_g1)
        sizes = [CHUNK_E] * full_chunks + ([tail_e] if tail_e else [])

        def idx_refs(k):
            ne = sizes[k]
            sl = pl.ds(0, ne)
            return row_r.at[k % 4].at[sl], col_r.at[k % 4].at[sl]

        def fire_idx(k):
            base = e0 + k * CHUNK_E
            ne = sizes[k]
            rv, cv = idx_refs(k)
            pltpu.async_copy(row_hbm.at[pl.ds(base, ne)], rv, sem_i)
            pltpu.async_copy(col_hbm.at[pl.ds(base, ne)], cv, sem_i)

        def wait_idx(k):
            ne = sizes[k]
            rv, cv = idx_refs(k)
            pltpu.make_async_copy(row_hbm.at[pl.ds(e0, ne)], rv,
                                  sem_i).wait()
            pltpu.make_async_copy(col_hbm.at[pl.ds(e0, ne)], cv,
                                  sem_i).wait()

        def fire_gather(k):
            ne = sizes[k]
            _, cv = idx_refs(k)
            return pltpu.async_copy(
                y_hbm.at[cv], rows_b[k % 2].at[pl.ds(0, ne)], gsem[k % 2])

        fire_idx(0)
        if n_bodies > 1:
            fire_idx(1)
        wait_idx(0)
        gds = {0: fire_gather(0)}
        sds = {}
        for k in range(n_bodies):
            p = k % 2
            ne = sizes[k]
            rv, _ = idx_refs(k)
            if k - 1 in sds:
                sds[k - 1].wait()
            if k + 1 < n_bodies:
                wait_idx(k + 1)
                gds[k + 1] = fire_gather(k + 1)
            if k + 2 < n_bodies:
                fire_idx(k + 2)
            @pl.loop(0, ne, step=16)
            def _(i, _k=k):
                plsc.addupdate_scatter(
                    hist_v, [row_r.at[_k % 4][pl.ds(i, 16)]], ones_reg)

            gds[k].wait()
            sds[k] = pltpu.async_copy(rows_b[p].at[pl.ds(0, ne)],
                                      nb_acc.at[rv], sem_s, add=True)
        sds[n_bodies - 1].wait()

        plsc.subcore_barrier()

        def writeout(dst, nrows):
            sl = pl.ds(r0, nrows)
            pltpu.sync_copy(nb_acc.at[sl], dst.at[ci].at[sl])

        def stage2(nrows):
            sl = pl.ds(r0, nrows)
            wd = pltpu.async_copy(nb_acc.at[sl], nb_out.at[ci].at[sl],
                                  sem_g0)
            hd = pltpu.async_copy(hist_v, cnt_out.at[wid], sem_g1)
            wd.wait()
            zd = zero_nb_async(nrows)
            hd.wait()
            zd.wait()

        @pl.when(si < NS - 1)
        def _():
            stage2(rsub)

        @pl.when(si == NS - 1)
        def _():
            stage2(rlast)

        plsc.subcore_barrier()

        @pl.when(wid < id_chunks)
        def _():
            base = wid * ID_CHUNK
            idv = row_r.at[0].at[pl.ds(0, ID_CHUNK)]
            gv = rows_v0.at[pl.ds(0, ID_CHUNK)]
            pltpu.sync_copy(ids_hbm.at[pl.ds(base, ID_CHUNK)], idv)
            pltpu.async_copy(zp_hbm.at[idv], gv, sem_g0).wait()
            pltpu.sync_copy(gv, nb_acc.at[idv], add=True)

        plsc.subcore_barrier()

        @pl.when(si < NS - 1)
        def _():
            writeout(idp_out, rsub)

        @pl.when(si == NS - 1)
        def _():
            writeout(idp_out, rlast)

    nb_p, cnt_p, idp_p = sc_scatter(y, zpad, row1d, col1d, ids1d, zeros64)

    out = pl.pallas_call(
        _combine_body,
        grid=(grid1,),
        in_specs=[
            pl.BlockSpec((blk, ku), lambda i: (i, 0)),
            pl.BlockSpec((NC, blk, ku), lambda i: (0, i, 0)),
            pl.BlockSpec((blk, NW), lambda i: (i, 0)),
            pl.BlockSpec((NC, blk, ku), lambda i: (0, i, 0)),
            pl.BlockSpec((1, 2 * ku), lambda i: (0, 0)),
        ],
        out_specs=pl.BlockSpec((blk, 2 * ku), lambda i: (i, 0)),
        out_shape=jax.ShapeDtypeStruct((n, 2 * ku), jnp.float32),
    )(h, nb_p, cnt_p.T, idp_p, bias.reshape(1, 2 * ku))
    return out


# speedup vs baseline: 3.1896x; 1.0624x over previous
import functools

import jax
import jax.numpy as jnp
from jax import lax
from jax.experimental import pallas as pl
from jax.experimental.pallas import tpu as pltpu
from jax.experimental.pallas import tpu_sc as plsc

NC = 2
NS = 16
NW = NC * NS

SLAB = 128
CHUNK_E = 576
ID_CHUNK = 512


def _proj_body(x_ref, wn_ref, wi_ref, ws_ref, y_ref, z_ref, h_ref):
    xb = x_ref[...]
    y_ref[...] = jnp.dot(xb, wn_ref[...], preferred_element_type=jnp.float32)
    z_ref[...] = jnp.dot(xb, wi_ref[...], preferred_element_type=jnp.float32)
    h_ref[...] = jnp.dot(xb, ws_ref[...], preferred_element_type=jnp.float32)


def _combine_body(h_ref, nb_ref, cnt_ref, idp_ref, bias_ref, out_ref):
    ku = h_ref.shape[1]
    blk = h_ref.shape[0]
    left = h_ref[...] + idp_ref[0] + idp_ref[1]
    cnt = jnp.sum(cnt_ref[...], axis=1)
    right = (nb_ref[0] + nb_ref[1]) / jnp.maximum(cnt[:, None], 1.0)
    bias = bias_ref[...]
    out_ref[:, 0:ku] = jax.nn.relu(left + bias[0, 0:ku])
    out_ref[:, ku:] = jax.nn.relu(right + bias[0, ku:])


def kernel(x, edge_index, id_index, W_self, W_id, W_neighbor, bias):
    n, d = x.shape
    ku = W_self.shape[1]
    e = edge_index.shape[1]
    nid = id_index.shape[0]

    tile_e = e // NW
    full_chunks = tile_e // CHUNK_E
    tail_e = tile_e - full_chunks * CHUNK_E
    n_bodies = full_chunks + (1 if tail_e else 0)
    npad = n + 8
    id_pad = -(-nid // ID_CHUNK) * ID_CHUNK
    id_chunks = id_pad // ID_CHUNK
    rsub = 8 * ((n // NS) // 8)
    rlast = n - rsub * (NS - 1)

    blk = 2000
    grid1 = n // blk
    y, z, h = pl.pallas_call(
        _proj_body,
        grid=(grid1,),
        in_specs=[
            pl.BlockSpec((blk, d), lambda i: (i, 0)),
            pl.BlockSpec((d, ku), lambda i: (0, 0)),
            pl.BlockSpec((d, ku), lambda i: (0, 0)),
            pl.BlockSpec((d, ku), lambda i: (0, 0)),
        ],
        out_specs=[
            pl.BlockSpec((blk, ku), lambda i: (i, 0)),
            pl.BlockSpec((blk, ku), lambda i: (i, 0)),
            pl.BlockSpec((blk, ku), lambda i: (i, 0)),
        ],
        out_shape=[
            jax.ShapeDtypeStruct((n, ku), jnp.float32),
            jax.ShapeDtypeStruct((n, ku), jnp.float32),
            jax.ShapeDtypeStruct((n, ku), jnp.float32),
        ],
    )(x, W_neighbor, W_id, W_self)

    edges = edge_index.astype(jnp.int32)
    idg = jnp.concatenate(
        [id_index.astype(jnp.int32),
         jnp.zeros((id_pad - nid,), dtype=jnp.int32)])
    idsc = jnp.concatenate(
        [id_index.astype(jnp.int32),
         jnp.full((id_pad - nid,), n, dtype=jnp.int32)])
    zeros64 = jnp.zeros((rlast, ku), jnp.float32)

    mesh = plsc.VectorSubcoreMesh(
        core_axis_name="c", subcore_axis_name="s",
        num_cores=NC, num_subcores=NS)

    @functools.partial(
        pl.kernel,
        out_type=(
            jax.ShapeDtypeStruct((NC, n, ku), jnp.float32),
            jax.ShapeDtypeStruct((NW, n), jnp.float32),
            jax.ShapeDtypeStruct((NC, n, ku), jnp.float32),
        ),
        mesh=mesh,
        compiler_params=pltpu.CompilerParams(
            use_tc_tiling_on_sc=False, needs_layout_passes=False),
        scratch_types=[
            pltpu.VMEM((4, CHUNK_E), jnp.int32),
            pltpu.VMEM((4, CHUNK_E), jnp.int32),
            pltpu.VMEM((CHUNK_E, ku), jnp.float32),
            pltpu.VMEM((CHUNK_E, ku), jnp.float32),
            pltpu.VMEM((n,), jnp.float32),
            pltpu.VMEM_SHARED((npad, ku), jnp.float32),
            pltpu.SemaphoreType.DMA,
            pltpu.SemaphoreType.DMA,
            pltpu.SemaphoreType.DMA,
            pltpu.SemaphoreType.DMA,
        ],
    )
    def sc_scatter(y_hbm, z_hbm, edge_hbm, idg_hbm, idsc_hbm, z64_hbm,
                   nb_out, cnt_out, idp_out,
                   row_r, col_r, rows_v0, rows_v1, hist_v,
                   nb_acc, sem_i, sem_g0, sem_g1, sem_s):
        ci = lax.axis_index("c")
        si = lax.axis_index("s")
        wid = si * NC + ci

        r0 = si * rsub

        def zero_nb_async(nrows):
            return pltpu.async_copy(z64_hbm.at[pl.ds(0, nrows)],
                                    nb_acc.at[pl.ds(r0, nrows)], sem_s)

        def zero_nb(nrows):
            zero_nb_async(nrows).wait()

        @pl.when(si < NS - 1)
        def _():
            zd = zero_nb_async(rsub)
            zv = jnp.zeros((16,), jnp.float32)

            @pl.loop(0, n, step=16)
            def _(i):
                hist_v[pl.ds(i, 16)] = zv

            zd.wait()

        @pl.when(si == NS - 1)
        def _():
            zd = zero_nb_async(rlast)
            zv = jnp.zeros((16,), jnp.float32)

            @pl.loop(0, n, step=16)
            def _(i):
                hist_v[pl.ds(i, 16)] = zv

            zd.wait()

        plsc.subcore_barrier()

        ones_reg = jnp.ones((16,), jnp.float32)
        e0 = wid * tile_e
        rows_b = (rows_v0, rows_v1)
        gsem = (sem_g0, sem_g1)
        sizes = [CHUNK_E] * full_chunks + ([tail_e] if tail_e else [])

        def idx_refs(k):
            ne = sizes[k]
            sl = pl.ds(0, ne)
            return row_r.at[k % 4].at[sl], col_r.at[k % 4].at[sl]

        def fire_idx(k):
            base = e0 + k * CHUNK_E
            ne = sizes[k]
            rv, cv = idx_refs(k)
            pltpu.async_copy(edge_hbm.at[0].at[pl.ds(base, ne)], rv, sem_i)
            pltpu.async_copy(edge_hbm.at[1].at[pl.ds(base, ne)], cv, sem_i)

        def wait_idx(k):
            ne = sizes[k]
            rv, cv = idx_refs(k)
            pltpu.make_async_copy(edge_hbm.at[0].at[pl.ds(e0, ne)], rv,
                                  sem_i).wait()
            pltpu.make_async_copy(edge_hbm.at[1].at[pl.ds(e0, ne)], cv,
                                  sem_i).wait()

        def fire_gather(k):
            ne = sizes[k]
            _, cv = idx_refs(k)
            return pltpu.async_copy(
                y_hbm.at[cv], rows_b[k % 2].at[pl.ds(0, ne)], gsem[k % 2])

        fire_idx(0)
        if n_bodies > 1:
            fire_idx(1)
        wait_idx(0)
        gds = {0: fire_gather(0)}
        sds = {}
        for k in range(n_bodies):
            p = k % 2
            ne = sizes[k]
            rv, _ = idx_refs(k)
            if k - 1 in sds:
                sds[k - 1].wait()
            if k + 1 < n_bodies:
                wait_idx(k + 1)
                gds[k + 1] = fire_gather(k + 1)
            if k + 2 < n_bodies:
                fire_idx(k + 2)
            @pl.loop(0, ne, step=16)
            def _(i, _k=k):
                plsc.addupdate_scatter(
                    hist_v, [row_r.at[_k % 4][pl.ds(i, 16)]], ones_reg)

            gds[k].wait()
            sds[k] = pltpu.async_copy(rows_b[p].at[pl.ds(0, ne)],
                                      nb_acc.at[rv], sem_s, add=True)
        sds[n_bodies - 1].wait()

        plsc.subcore_barrier()

        def writeout(dst, nrows):
            sl = pl.ds(r0, nrows)
            pltpu.sync_copy(nb_acc.at[sl], dst.at[ci].at[sl])

        def stage2(nrows):
            sl = pl.ds(r0, nrows)
            wd = pltpu.async_copy(nb_acc.at[sl], nb_out.at[ci].at[sl],
                                  sem_g0)
            hd = pltpu.async_copy(hist_v, cnt_out.at[wid], sem_g1)
            wd.wait()
            zd = zero_nb_async(nrows)
            hd.wait()
            zd.wait()

        @pl.when(si < NS - 1)
        def _():
            stage2(rsub)

        @pl.when(si == NS - 1)
        def _():
            stage2(rlast)

        plsc.subcore_barrier()

        @pl.when(wid < id_chunks)
        def _():
            base = wid * ID_CHUNK
            idv = row_r.at[0].at[pl.ds(0, ID_CHUNK)]
            isv = row_r.at[1].at[pl.ds(0, ID_CHUNK)]
            gv = rows_v0.at[pl.ds(0, ID_CHUNK)]
            pltpu.sync_copy(idg_hbm.at[pl.ds(base, ID_CHUNK)], idv)
            pltpu.sync_copy(idsc_hbm.at[pl.ds(base, ID_CHUNK)], isv)
            pltpu.async_copy(z_hbm.at[idv], gv, sem_g0).wait()
            pltpu.sync_copy(gv, nb_acc.at[isv], add=True)

        plsc.subcore_barrier()

        @pl.when(si < NS - 1)
        def _():
            writeout(idp_out, rsub)

        @pl.when(si == NS - 1)
        def _():
            writeout(idp_out, rlast)

    nb_p, cnt_p, idp_p = sc_scatter(y, z, edges, idg, idsc, zeros64)

    out = pl.pallas_call(
        _combine_body,
        grid=(grid1,),
        in_specs=[
            pl.BlockSpec((blk, ku), lambda i: (i, 0)),
            pl.BlockSpec((NC, blk, ku), lambda i: (0, i, 0)),
            pl.BlockSpec((blk, NW), lambda i: (i, 0)),
            pl.BlockSpec((NC, blk, ku), lambda i: (0, i, 0)),
            pl.BlockSpec((1, 2 * ku), lambda i: (0, 0)),
        ],
        out_specs=pl.BlockSpec((blk, 2 * ku), lambda i: (i, 0)),
        out_shape=jax.ShapeDtypeStruct((n, 2 * ku), jnp.float32),
    )(h, nb_p, cnt_p.T, idp_p, bias.reshape(1, 2 * ku))
    return out
